# baseline (speedup 1.0000x reference)
import functools

import jax
import jax.numpy as jnp
from jax import lax
from jax.experimental import pallas as pl
from jax.experimental.pallas import tpu as pltpu
from jax.experimental.pallas import tpu_sc as plsc

_NC = 2
_NS = 16
_LANES = 16
_NW = _NC * _NS
_W = 256


@functools.lru_cache(maxsize=None)
def _build_sc_call(B, L, D):
    F = L * D
    g_per = F // _NW
    n_out = B // _W
    mesh = plsc.VectorSubcoreMesh(core_axis_name="c", subcore_axis_name="s")

    @functools.partial(
        pl.kernel,
        out_type=jax.ShapeDtypeStruct((F, B), jnp.float32),
        mesh=mesh,
        scratch_types=[
            pltpu.VMEM((g_per,), jnp.float32),
            pltpu.VMEM((g_per,), jnp.float32),
            pltpu.VMEM((g_per,), jnp.float32),
            pltpu.VMEM((g_per, _W), jnp.float32),
            pltpu.SemaphoreType.DMA,
        ],
    )
    def sc_fn(emb_hbm, pos_hbm, out_hbm, ebuf, pbuf, cbuf, rbuf, sem):
        wid = lax.axis_index("s") * _NC + lax.axis_index("c")
        gbase = wid * g_per
        pltpu.sync_copy(emb_hbm.at[pl.ds(gbase, g_per)], ebuf)
        pltpu.sync_copy(pos_hbm.at[pl.ds(gbase, g_per)], pbuf)

        def add_body(i, carry):
            sl = pl.ds(pl.multiple_of(i * _LANES, _LANES), _LANES)
            cbuf[sl] = ebuf[sl] + pbuf[sl]
            return carry

        lax.fori_loop(0, g_per // _LANES, add_body, 0)

        def blk_body(i, carry):
            sl = pl.ds(pl.multiple_of(i * _LANES, _LANES), _LANES)
            v16 = cbuf[sl]
            for j in range(_LANES):
                w = v16.at[jnp.full((_LANES,), j, jnp.int32)].get(
                    mode="promise_in_bounds"
                )
                row = rbuf.at[i * _LANES + j]
                for c in range(_W // _LANES):
                    row[pl.ds(c * _LANES, _LANES)] = w
            return carry

        lax.fori_loop(0, g_per // _LANES, blk_body, 0)

        copies = [
            pltpu.async_copy(
                rbuf,
                out_hbm.at[pl.ds(gbase, g_per), pl.ds(t * _W, _W)],
                sem,
            )
            for t in range(n_out)
        ]
        for c in copies:
            c.wait()

    return sc_fn


def kernel(input_char, emb_table, pos_table):
    B, L = input_char.shape
    D = emb_table.shape[1]
    emb_flat = emb_table[:L].reshape(L * D)
    pos_flat = pos_table[0, :L].reshape(L * D)
    out_fb = _build_sc_call(B, L, D)(emb_flat, pos_flat)
    return jnp.transpose(out_fb, (1, 0)).reshape(B, L, D)


# speedup vs baseline: 1.0283x; 1.0283x over previous
import functools

import jax
import jax.numpy as jnp
from jax import lax
from jax.experimental import pallas as pl
from jax.experimental.pallas import tpu as pltpu
from jax.experimental.pallas import tpu_sc as plsc

_NC = 2
_NS = 16
_LANES = 16
_NW = _NC * _NS
_W = 128


@functools.lru_cache(maxsize=None)
def _build_sc_call(B, L, D):
    F = L * D
    g_per = F // _NW
    n_out = B // _W
    mesh = plsc.VectorSubcoreMesh(core_axis_name="c", subcore_axis_name="s")

    @functools.partial(
        pl.kernel,
        out_type=jax.ShapeDtypeStruct((F, B), jnp.float32),
        mesh=mesh,
        scratch_types=[
            pltpu.VMEM((g_per,), jnp.float32),
            pltpu.VMEM((g_per,), jnp.float32),
            pltpu.VMEM((g_per, _W), jnp.float32),
            pltpu.SemaphoreType.DMA,
        ],
    )
    def sc_fn(emb_hbm, pos_hbm, out_hbm, ebuf, pbuf, rbuf, sem):
        wid = lax.axis_index("s") * _NC + lax.axis_index("c")
        gbase = wid * g_per
        in_copies = [
            pltpu.async_copy(emb_hbm.at[pl.ds(gbase, g_per)], ebuf, sem),
            pltpu.async_copy(pos_hbm.at[pl.ds(gbase, g_per)], pbuf, sem),
        ]
        for c in in_copies:
            c.wait()

        def blk_body(i, carry):
            sl = pl.ds(pl.multiple_of(i * _LANES, _LANES), _LANES)
            v16 = ebuf[sl] + pbuf[sl]
            for j in range(_LANES):
                w = v16.at[jnp.full((_LANES,), j, jnp.int32)].get(
                    mode="promise_in_bounds"
                )
                row = rbuf.at[i * _LANES + j]
                for c in range(_W // _LANES):
                    row[pl.ds(c * _LANES, _LANES)] = w
            return carry

        lax.fori_loop(0, g_per // _LANES, blk_body, 0)

        copies = [
            pltpu.async_copy(
                rbuf,
                out_hbm.at[pl.ds(gbase, g_per), pl.ds(t * _W, _W)],
                sem,
            )
            for t in range(n_out)
        ]
        for c in copies:
            c.wait()

    return sc_fn


def kernel(input_char, emb_table, pos_table):
    B, L = input_char.shape
    D = emb_table.shape[1]
    emb_flat = emb_table[:L].reshape(L * D)
    pos_flat = pos_table[0, :L].reshape(L * D)
    out_fb = _build_sc_call(B, L, D)(emb_flat, pos_flat)
    return jnp.transpose(out_fb, (1, 0)).reshape(B, L, D)


# speedup vs baseline: 1.0313x; 1.0029x over previous
import functools

import jax
import jax.numpy as jnp
from jax import lax
from jax.experimental import pallas as pl
from jax.experimental.pallas import tpu as pltpu
from jax.experimental.pallas import tpu_sc as plsc

_NC = 2
_NS = 16
_LANES = 16
_NW = _NC * _NS
_W = 128


@functools.lru_cache(maxsize=None)
def _build_sc_call(B, L, D):
    F = L * D
    g_per = F // _NW
    n_out = B // _W
    mesh = plsc.VectorSubcoreMesh(core_axis_name="c", subcore_axis_name="s")

    @functools.partial(
        pl.kernel,
        out_type=jax.ShapeDtypeStruct((F, B), jnp.float32),
        mesh=mesh,
        scratch_types=[
            pltpu.VMEM((g_per,), jnp.float32),
            pltpu.VMEM((g_per,), jnp.float32),
            pltpu.VMEM((g_per, _W), jnp.float32),
            pltpu.SemaphoreType.DMA,
        ],
    )
    def sc_fn(emb_hbm, pos_hbm, out_hbm, ebuf, pbuf, rbuf, sem):
        wid = lax.axis_index("s") * _NC + lax.axis_index("c")
        gbase = wid * g_per
        in_copies = [
            pltpu.async_copy(emb_hbm.at[pl.ds(gbase, g_per)], ebuf, sem),
            pltpu.async_copy(pos_hbm.at[pl.ds(gbase, g_per)], pbuf, sem),
        ]
        for c in in_copies:
            c.wait()

        def blk_body(i, carry):
            sl = pl.ds(pl.multiple_of(i * _LANES, _LANES), _LANES)
            v16 = ebuf[sl] + pbuf[sl]
            for j in range(_LANES):
                w = v16.at[jnp.full((_LANES,), j, jnp.int32)].get(
                    mode="promise_in_bounds"
                )
                row = rbuf.at[i * _LANES + j]
                for c in range(_W // _LANES):
                    row[pl.ds(c * _LANES, _LANES)] = w
            return carry

        half = g_per // 2
        copies = []
        for p in range(2):
            lax.fori_loop(p * half // _LANES, (p + 1) * half // _LANES,
                          blk_body, 0)
            copies.extend(
                pltpu.async_copy(
                    rbuf.at[pl.ds(p * half, half)],
                    out_hbm.at[pl.ds(gbase + p * half, half),
                               pl.ds(t * _W, _W)],
                    sem,
                )
                for t in range(n_out)
            )
        for c in copies:
            c.wait()

    return sc_fn


def kernel(input_char, emb_table, pos_table):
    B, L = input_char.shape
    D = emb_table.shape[1]
    emb_flat = emb_table[:L].reshape(L * D)
    pos_flat = pos_table[0, :L].reshape(L * D)
    out_fb = _build_sc_call(B, L, D)(emb_flat, pos_flat)
    return jnp.transpose(out_fb, (1, 0)).reshape(B, L, D)
